# baseline (speedup 1.0000x reference)
import functools

import jax
import jax.numpy as jnp
from jax import lax
from jax.experimental import pallas as pl
from jax.experimental.pallas import tpu as pltpu
from jax.experimental.pallas import tpu_sc as plsc

_N = 10000
_E = 320000
_NC = 2
_NS = 16
_CH = 80
_EPAD = 327680
_NP = 10240
_RPT = _NP // _NS


@functools.lru_cache(maxsize=None)
def _make_segsum():
    ntile = _NC * _NS
    ept = _EPAD // ntile
    nch = ept // _CH
    nbuf = 4
    mesh = plsc.VectorSubcoreMesh(core_axis_name="c", subcore_axis_name="s")

    nidx = 2 * nbuf
    @functools.partial(
        pl.kernel,
        out_type=jax.ShapeDtypeStruct((_NC, _NP, 128), jnp.float32),
        mesh=mesh,
        scratch_types=[
            [pltpu.VMEM((2, _CH), jnp.int32) for _ in range(nidx)],
            [pltpu.VMEM((_CH, 128), jnp.float32) for _ in range(nbuf)],
            pltpu.VMEM_SHARED((_NP, 128), jnp.float32),
            [pltpu.SemaphoreType.DMA for _ in range(nidx)],
            [pltpu.SemaphoreType.DMA for _ in range(nbuf)],
            [pltpu.SemaphoreType.DMA for _ in range(nbuf)],
        ],
    )
    def segsum(xcat, idx3, zeros, out, ibuf, bufs, acc, isem, gsem, ssem):
        c = lax.axis_index("c")
        s = lax.axis_index("s")
        w = c * _NS + s

        pltpu.sync_copy(zeros.at[pl.ds(s * _RPT, _RPT)],
                        acc.at[pl.ds(s * _RPT, _RPT)])

        def istart(ch, k):
            pltpu.async_copy(idx3.at[w * nch + ch], ibuf[k], isem[k])

        def iwait(k):
            pltpu.make_async_copy(idx3.at[0], ibuf[k], isem[k]).wait()

        def gstart(kb, ki):
            pltpu.async_copy(xcat.at[ibuf[ki].at[0]], bufs[kb], gsem[kb])

        def gwait(kb, ki):
            pltpu.make_async_copy(xcat.at[ibuf[ki].at[0]], bufs[kb],
                                  gsem[kb]).wait()

        def sstart(kb, ki):
            pltpu.async_copy(bufs[kb], acc.at[ibuf[ki].at[1]], ssem[kb],
                             add=True)

        def swait(kb, ki):
            pltpu.make_async_copy(bufs[kb], acc.at[ibuf[ki].at[1]],
                                  ssem[kb]).wait()

        plsc.subcore_barrier()

        for j in range(nidx):
            istart(j, j)
        for j in range(2):
            iwait(j)
            gstart(j, j)

        def body(g, _):
            for kk in range(nidx):
                ch = nidx * g + kk
                kb = kk % nbuf
                ki = kk
                gwait(kb, ki)
                sstart(kb, ki)
                kb2 = (kb + 2) % nbuf
                ki2 = (ki + 2) % nidx
                ki_old = (ki + 2 + nbuf) % nidx

                @pl.when(ch + 2 < nch)
                def _():
                    @pl.when(ch >= 2)
                    def _():
                        swait(kb2, ki_old)

                        @pl.when(ch + 6 < nch)
                        def _():
                            istart(ch + 6, ki_old)

                    iwait(ki2)
                    gstart(kb2, ki2)
            return 0

        lax.fori_loop(0, nch // nidx, body, 0)

        for j in range(nbuf):
            ch = nch - nbuf + j
            swait(ch % nbuf, ch % nidx)

        plsc.subcore_barrier()
        pltpu.sync_copy(acc.at[pl.ds(s * _RPT, _RPT)],
                        out.at[c, pl.ds(s * _RPT, _RPT)])

    return segsum


_BR = 1000
_G = _N // _BR


def _stats_update(i, st_ref, t):
    @pl.when(i == 0)
    def _():
        st_ref[...] = jnp.zeros_like(st_ref)

    st_ref[...] += jnp.stack([jnp.sum(t, axis=0), jnp.sum(t * t, axis=0)])


def _bn_coefs(st_ref, g, b):
    m = st_ref[0:1, :] * (1.0 / _N)
    v = st_ref[1:2, :] * (1.0 / _N) - m * m
    r = lax.rsqrt(v + 1e-5)
    return r * g, b - m * r * g


def _tc_layer(x, aggs, scale, w0, b0, g0, be0, w1, b1, ga, ba, go, bo,
              emit_cat):
    n, d = x.shape
    do = w1.shape[1]
    dh = do // 2
    na = len(aggs)

    def body(scale_ref, x_ref, *rest):
        agg_refs = rest[:na]
        (w0_ref, b0_ref, g0_ref, be0_ref, w1_ref, b1_ref, ga_ref, ba_ref,
         go_ref, bo_ref) = rest[na:na + 10]
        outs = rest[na + 10:]
        x_out = outs[0]
        t_s = outs[-2]
        st_s = outs[-1]
        p = pl.program_id(0)
        i = pl.program_id(1)
        blk = pl.ds(i * _BR, _BR)

        def stats(k, t):
            @pl.when(i == 0)
            def _():
                st_s[k] = jnp.zeros((2, do), jnp.float32)

            st_s[k] += jnp.stack([jnp.sum(t, axis=0), jnp.sum(t * t, axis=0)])

        def coefs(k, g, b):
            st = st_s[k]
            m = st[0:1, :] * (1.0 / _N)
            v = st[1:2, :] * (1.0 / _N) - m * m
            r = lax.rsqrt(v + 1e-5)
            return r * g, b - m * r * g

        @pl.when(p == 0)
        def _():
            chunks = [a[0] + a[1] for a in agg_refs]
            agg = chunks[0] if na == 1 else jnp.concatenate(chunks, axis=1)
            rst = scale_ref[0, 0] * x_ref[...] + agg
            t = jnp.dot(rst, w0_ref[...],
                        preferred_element_type=jnp.float32) + b0_ref[...]
            t_s[blk] = t
            stats(0, t)

        @pl.when(p == 1)
        def _():
            a, cc = coefs(0, g0_ref[...], be0_ref[...])
            u = jnp.maximum(t_s[blk] * a + cc, 0.0)
            t2 = jnp.dot(u, w1_ref[...],
                         preferred_element_type=jnp.float32) + b1_ref[...]
            t_s[blk] = t2
            stats(1, t2)

        @pl.when(p == 2)
        def _():
            a, cc = coefs(1, ga_ref[...], ba_ref[...])
            v = jnp.maximum(t_s[blk] * a + cc, 0.0)
            t_s[blk] = v
            stats(2, v)

        @pl.when(p == 3)
        def _():
            a, cc = coefs(2, go_ref[...], bo_ref[...])
            xo = jnp.maximum(t_s[blk] * a + cc, 0.0)
            x_out[...] = xo
            if emit_cat:
                cat_ref = outs[1]
                cat_ref[0] = xo[:, :dh]
                cat_ref[1] = xo[:, dh:]

    out_specs = [
        pl.BlockSpec((_BR, do), lambda p, i: (jnp.where(p == 3, i, 0), 0)),
    ]
    out_shape = [jax.ShapeDtypeStruct((n, do), jnp.float32)]
    if emit_cat:
        out_specs.append(
            pl.BlockSpec((2, _BR, dh),
                         lambda p, i: (0, jnp.where(p == 3, i, 0), 0)))
        out_shape.append(jax.ShapeDtypeStruct((2, n, dh), jnp.float32))

    vec = lambda a: a.reshape(1, -1)
    return pl.pallas_call(
        body,
        grid=(4, _G),
        in_specs=[
            pl.BlockSpec(memory_space=pltpu.SMEM),
            pl.BlockSpec((_BR, d), lambda p, i: (jnp.where(p == 0, i, 0), 0)),
        ] + [
            pl.BlockSpec((2, _BR, 128),
                         lambda p, i: (0, jnp.where(p == 0, i, 0), 0))
            for _ in aggs
        ] + [
            pl.BlockSpec((d, do), lambda p, i: (0, 0)),
        ] + [
            pl.BlockSpec((1, do), lambda p, i: (0, 0)),
        ] * 3 + [
            pl.BlockSpec((do, do), lambda p, i: (0, 0)),
        ] + [
            pl.BlockSpec((1, do), lambda p, i: (0, 0)),
        ] * 5,
        out_specs=out_specs,
        out_shape=out_shape,
        scratch_shapes=[
            pltpu.VMEM((_N, do), jnp.float32),
            pltpu.VMEM((3, 2, do), jnp.float32),
        ],
    )(scale, x, *aggs, w0, vec(b0), vec(g0), vec(be0), w1, vec(b1), vec(ga),
      vec(ba), vec(go), vec(bo))


def kernel(h, edge_index,
           l0_W0, l0_b0, l0_g0, l0_be0, l0_W1, l0_b1, l0_ga, l0_ba, l0_go,
           l0_bo, l0_eps,
           l1_W0, l1_b0, l1_g0, l1_be0, l1_W1, l1_b1, l1_ga, l1_ba, l1_go,
           l1_bo, l1_eps):
    pad = _EPAD - _E
    srcp = jnp.concatenate([edge_index[0], jnp.zeros((pad,), jnp.int32)])
    dstp = jnp.concatenate(
        [edge_index[1],
         _N + (jnp.arange(pad, dtype=jnp.int32) % (_NP - _N))])
    idx3 = jnp.stack(
        [srcp.reshape(-1, _CH), dstp.reshape(-1, _CH)], axis=1)

    params = [
        (l0_W0, l0_b0, l0_g0, l0_be0, l0_W1, l0_b1, l0_ga, l0_ba, l0_go,
         l0_bo, l0_eps),
        (l1_W0, l1_b0, l1_g0, l1_be0, l1_W1, l1_b1, l1_ga, l1_ba, l1_go,
         l1_bo, l1_eps),
    ]

    outs = [h]
    x = h
    tables = [h]
    zeros = jnp.zeros((_NP, 128), jnp.float32)
    for i, (w0, b0, g0, be0, w1, b1, ga, ba, go, bo, eps) in enumerate(params):
        aggs = [_make_segsum()(t, idx3, zeros) for t in tables]
        scale = (1.0 + eps).reshape(1, 1)
        if i == 0:
            x, cat = _tc_layer(x, aggs, scale, w0, b0, g0, be0, w1, b1, ga,
                               ba, go, bo, emit_cat=True)
            tables = [cat[0], cat[1]]
        else:
            (x,) = _tc_layer(x, aggs, scale, w0, b0, g0, be0, w1, b1, ga,
                             ba, go, bo, emit_cat=False)
        outs.append(x)

    return tuple(outs)


# speedup vs baseline: 3.3891x; 3.3891x over previous
import functools

import jax
import jax.numpy as jnp
from jax import lax
from jax.experimental import pallas as pl
from jax.experimental.pallas import tpu as pltpu
from jax.experimental.pallas import tpu_sc as plsc

_N = 10000
_E = 320000
_NC = 2
_NS = 16
_CH = 80
_NP = 10240
_RPT = _NP // _NS


@functools.lru_cache(maxsize=None)
def _make_segsum():
    ntile = _NC * _NS
    ept = _E // ntile
    nch = ept // _CH
    half = nch // 2
    tail = nch - 2 * half
    mesh = plsc.VectorSubcoreMesh(core_axis_name="c", subcore_axis_name="s")

    @functools.partial(
        pl.kernel,
        out_type=jax.ShapeDtypeStruct((_NC, _NP, 128), jnp.float32),
        mesh=mesh,
        scratch_types=[
            pltpu.VMEM((ept,), jnp.int32),
            pltpu.VMEM((nch, _CH), jnp.int32),
            pltpu.VMEM((_CH, 128), jnp.float32),
            pltpu.VMEM((_CH, 128), jnp.float32),
            pltpu.VMEM_SHARED((_NP, 128), jnp.float32),
            pltpu.SemaphoreType.DMA,
            pltpu.SemaphoreType.DMA,
        ],
    )
    def segsum(xcat, srch, dsth, zeros, out, srcv, dstv, rows0, rows1, acc,
               sem0, sem1):
        c = lax.axis_index("c")
        s = lax.axis_index("s")
        w = c * _NS + s

        pltpu.sync_copy(zeros.at[pl.ds(s * _RPT, _RPT)],
                        acc.at[pl.ds(s * _RPT, _RPT)])

        pltpu.sync_copy(srch.at[pl.ds(w * ept, ept)], srcv)
        pltpu.sync_copy(dsth.at[w], dstv)

        plsc.subcore_barrier()

        def gstart(ch, buf, sem):
            pltpu.async_copy(xcat.at[srcv.at[pl.ds(ch * _CH, _CH)]], buf, sem)

        def gwait(buf, sem):
            pltpu.make_async_copy(xcat.at[srcv.at[pl.ds(0, _CH)]], buf,
                                  sem).wait()

        def scat(ch, buf):
            pltpu.sync_copy(buf, acc.at[dstv.at[ch]], add=True)

        gstart(0, rows0, sem0)

        def body(g, _):
            c0 = 2 * g
            gstart(c0 + 1, rows1, sem1)
            gwait(rows0, sem0)
            scat(c0, rows0)

            @pl.when(g < half - 1 + tail)
            def _():
                gstart(c0 + 2, rows0, sem0)

            gwait(rows1, sem1)
            scat(c0 + 1, rows1)
            return 0

        lax.fori_loop(0, half, body, 0)

        if tail:
            gwait(rows0, sem0)
            scat(nch - 1, rows0)

        plsc.subcore_barrier()
        pltpu.sync_copy(acc.at[pl.ds(s * _RPT, _RPT)],
                        out.at[c, pl.ds(s * _RPT, _RPT)])

    return segsum


_BR = 2000
_G = _N // _BR


def _stats_update(i, st_ref, t):
    @pl.when(i == 0)
    def _():
        st_ref[...] = jnp.zeros_like(st_ref)

    st_ref[...] += jnp.stack([jnp.sum(t, axis=0), jnp.sum(t * t, axis=0)])


def _bn_coefs(st_ref, g, b):
    m = st_ref[0:1, :] * (1.0 / _N)
    v = st_ref[1:2, :] * (1.0 / _N) - m * m
    r = lax.rsqrt(v + 1e-5)
    return r * g, b - m * r * g


def _tc_layer(x, aggs, scale, w0, b0, g0, be0, w1, b1, ga, ba, go, bo,
              emit_cat):
    n, d = x.shape
    do = w1.shape[1]
    dh = do // 2
    na = len(aggs)

    def body(scale_ref, x_ref, *rest):
        agg_refs = rest[:na]
        (w0_ref, b0_ref, g0_ref, be0_ref, w1_ref, b1_ref, ga_ref, ba_ref,
         go_ref, bo_ref) = rest[na:na + 10]
        outs = rest[na + 10:]
        x_out = outs[0]
        t_s = outs[-2]
        st_s = outs[-1]
        p = pl.program_id(0)
        i = pl.program_id(1)
        blk = pl.ds(i * _BR, _BR)

        def stats(k, t):
            @pl.when(i == 0)
            def _():
                st_s[k] = jnp.zeros((2, do), jnp.float32)

            st_s[k] += jnp.stack([jnp.sum(t, axis=0), jnp.sum(t * t, axis=0)])

        def coefs(k, g, b):
            st = st_s[k]
            m = st[0:1, :] * (1.0 / _N)
            v = st[1:2, :] * (1.0 / _N) - m * m
            r = lax.rsqrt(v + 1e-5)
            return r * g, b - m * r * g

        @pl.when(p == 0)
        def _():
            chunks = [a[0] + a[1] for a in agg_refs]
            agg = chunks[0] if na == 1 else jnp.concatenate(chunks, axis=1)
            rst = scale_ref[0, 0] * x_ref[...] + agg
            t = jnp.dot(rst, w0_ref[...],
                        preferred_element_type=jnp.float32) + b0_ref[...]
            t_s[blk] = t
            stats(0, t)

        @pl.when(p == 1)
        def _():
            a, cc = coefs(0, g0_ref[...], be0_ref[...])
            u = jnp.maximum(t_s[blk] * a + cc, 0.0)
            t2 = jnp.dot(u, w1_ref[...],
                         preferred_element_type=jnp.float32) + b1_ref[...]
            t_s[blk] = t2
            stats(1, t2)

        @pl.when(p == 2)
        def _():
            a, cc = coefs(1, ga_ref[...], ba_ref[...])
            v = jnp.maximum(t_s[blk] * a + cc, 0.0)
            t_s[blk] = v
            stats(2, v)

        @pl.when(p == 3)
        def _():
            a, cc = coefs(2, go_ref[...], bo_ref[...])
            xo = jnp.maximum(t_s[blk] * a + cc, 0.0)
            x_out[...] = xo
            if emit_cat:
                cat_ref = outs[1]
                cat_ref[0] = xo[:, :dh]
                cat_ref[1] = xo[:, dh:]

    out_specs = [
        pl.BlockSpec((_BR, do), lambda p, i: (jnp.where(p == 3, i, 0), 0)),
    ]
    out_shape = [jax.ShapeDtypeStruct((n, do), jnp.float32)]
    if emit_cat:
        out_specs.append(
            pl.BlockSpec((2, _BR, dh),
                         lambda p, i: (0, jnp.where(p == 3, i, 0), 0)))
        out_shape.append(jax.ShapeDtypeStruct((2, n, dh), jnp.float32))

    vec = lambda a: a.reshape(1, -1)
    return pl.pallas_call(
        body,
        grid=(4, _G),
        in_specs=[
            pl.BlockSpec(memory_space=pltpu.SMEM),
            pl.BlockSpec((_BR, d), lambda p, i: (jnp.where(p == 0, i, 0), 0)),
        ] + [
            pl.BlockSpec((2, _BR, 128),
                         lambda p, i: (0, jnp.where(p == 0, i, 0), 0))
            for _ in aggs
        ] + [
            pl.BlockSpec((d, do), lambda p, i: (0, 0)),
        ] + [
            pl.BlockSpec((1, do), lambda p, i: (0, 0)),
        ] * 3 + [
            pl.BlockSpec((do, do), lambda p, i: (0, 0)),
        ] + [
            pl.BlockSpec((1, do), lambda p, i: (0, 0)),
        ] * 5,
        out_specs=out_specs,
        out_shape=out_shape,
        scratch_shapes=[
            pltpu.VMEM((_N, do), jnp.float32),
            pltpu.VMEM((3, 2, do), jnp.float32),
        ],
    )(scale, x, *aggs, w0, vec(b0), vec(g0), vec(be0), w1, vec(b1), vec(ga),
      vec(ba), vec(go), vec(bo))


def kernel(h, edge_index,
           l0_W0, l0_b0, l0_g0, l0_be0, l0_W1, l0_b1, l0_ga, l0_ba, l0_go,
           l0_bo, l0_eps,
           l1_W0, l1_b0, l1_g0, l1_be0, l1_W1, l1_b1, l1_ga, l1_ba, l1_go,
           l1_bo, l1_eps):
    src = edge_index[0]
    dst = edge_index[1]

    params = [
        (l0_W0, l0_b0, l0_g0, l0_be0, l0_W1, l0_b1, l0_ga, l0_ba, l0_go,
         l0_bo, l0_eps),
        (l1_W0, l1_b0, l1_g0, l1_be0, l1_W1, l1_b1, l1_ga, l1_ba, l1_go,
         l1_bo, l1_eps),
    ]

    outs = [h]
    x = h
    tables = [h]
    zeros = jnp.zeros((_NP, 128), jnp.float32)
    ntile = _NC * _NS
    dst3 = dst.reshape(ntile, _E // ntile // _CH, _CH)

    for i, (w0, b0, g0, be0, w1, b1, ga, ba, go, bo, eps) in enumerate(params):
        aggs = [_make_segsum()(t, src, dst3, zeros) for t in tables]
        scale = (1.0 + eps).reshape(1, 1)
        if i == 0:
            x, cat = _tc_layer(x, aggs, scale, w0, b0, g0, be0, w1, b1, ga,
                               ba, go, bo, emit_cat=True)
            tables = [cat[0], cat[1]]
        else:
            (x,) = _tc_layer(x, aggs, scale, w0, b0, g0, be0, w1, b1, ga,
                             ba, go, bo, emit_cat=False)
        outs.append(x)

    return tuple(outs)


# speedup vs baseline: 3.4330x; 1.0130x over previous
import functools

import jax
import jax.numpy as jnp
from jax import lax
from jax.experimental import pallas as pl
from jax.experimental.pallas import tpu as pltpu
from jax.experimental.pallas import tpu_sc as plsc

_N = 10000
_E = 320000
_NC = 2
_NS = 16
_CH = 80
_NP = 10240
_RPT = _NP // _NS


@functools.lru_cache(maxsize=None)
def _make_segsum():
    ntile = _NC * _NS
    ept = _E // ntile
    nch = ept // _CH
    half = nch // 2
    tail = nch - 2 * half
    mesh = plsc.VectorSubcoreMesh(core_axis_name="c", subcore_axis_name="s")

    @functools.partial(
        pl.kernel,
        out_type=jax.ShapeDtypeStruct((_NC, _NP, 128), jnp.float32),
        mesh=mesh,
        scratch_types=[
            pltpu.VMEM((ept,), jnp.int32),
            pltpu.VMEM((nch, _CH), jnp.int32),
            pltpu.VMEM((_CH, 128), jnp.float32),
            pltpu.VMEM((_CH, 128), jnp.float32),
            pltpu.VMEM_SHARED((_NP, 128), jnp.float32),
            pltpu.SemaphoreType.DMA,
            pltpu.SemaphoreType.DMA,
        ],
    )
    def segsum(xcat, srch, dsth, zeros, out, srcv, dstv, rows0, rows1, acc,
               sem0, sem1):
        c = lax.axis_index("c")
        s = lax.axis_index("s")
        w = c * _NS + s

        pltpu.sync_copy(zeros.at[pl.ds(s * _RPT, _RPT)],
                        acc.at[pl.ds(s * _RPT, _RPT)])

        pltpu.sync_copy(srch.at[pl.ds(w * ept, ept)], srcv)
        pltpu.sync_copy(dsth.at[w], dstv)

        plsc.subcore_barrier()

        def gstart(ch, buf, sem):
            pltpu.async_copy(xcat.at[srcv.at[pl.ds(ch * _CH, _CH)]], buf, sem)

        def gwait(buf, sem):
            pltpu.make_async_copy(xcat.at[srcv.at[pl.ds(0, _CH)]], buf,
                                  sem).wait()

        def scat(ch, buf):
            pltpu.sync_copy(buf, acc.at[dstv.at[ch]], add=True)

        gstart(0, rows0, sem0)

        def body(g, _):
            c0 = 2 * g
            gstart(c0 + 1, rows1, sem1)
            gwait(rows0, sem0)
            scat(c0, rows0)

            @pl.when(g < half - 1 + tail)
            def _():
                gstart(c0 + 2, rows0, sem0)

            gwait(rows1, sem1)
            scat(c0 + 1, rows1)
            return 0

        lax.fori_loop(0, half, body, 0)

        if tail:
            gwait(rows0, sem0)
            scat(nch - 1, rows0)

        plsc.subcore_barrier()
        pltpu.sync_copy(acc.at[pl.ds(s * _RPT, _RPT)],
                        out.at[c, pl.ds(s * _RPT, _RPT)])

    return segsum


_BR = 2000
_G = _N // _BR


def _stats_update(i, st_ref, t):
    @pl.when(i == 0)
    def _():
        st_ref[...] = jnp.zeros_like(st_ref)

    st_ref[...] += jnp.stack([jnp.sum(t, axis=0), jnp.sum(t * t, axis=0)])


def _bn_coefs(st_ref, g, b):
    m = st_ref[0:1, :] * (1.0 / _N)
    v = st_ref[1:2, :] * (1.0 / _N) - m * m
    r = lax.rsqrt(v + 1e-5)
    return r * g, b - m * r * g


def _tc_layer(x, aggs, scale, w0, b0, g0, be0, w1, b1, ga, ba, go, bo,
              emit_cat):
    n, d = x.shape
    do = w1.shape[1]
    dh = do // 2
    na = len(aggs)

    def body(scale_ref, x_ref, *rest):
        agg_refs = rest[:na]
        (w0_ref, b0_ref, g0_ref, be0_ref, w1_ref, b1_ref, ga_ref, ba_ref,
         go_ref, bo_ref) = rest[na:na + 10]
        outs = rest[na + 10:]
        x_out = outs[0]
        t_s = outs[-2]
        st_s = outs[-1]
        p = pl.program_id(0)
        i = pl.program_id(1)
        blk = pl.ds(i * _BR, _BR)

        def stats(k, t):
            @pl.when(i == 0)
            def _():
                st_s[k] = jnp.zeros((2, do), jnp.float32)

            st_s[k] += jnp.stack([jnp.sum(t, axis=0), jnp.sum(t * t, axis=0)])

        def coefs(k, g, b):
            st = st_s[k]
            m = st[0:1, :] * (1.0 / _N)
            v = st[1:2, :] * (1.0 / _N) - m * m
            r = lax.rsqrt(v + 1e-5)
            return r * g, b - m * r * g

        @pl.when(p == 0)
        def _():
            chunks = [a[0] + a[1] for a in agg_refs]
            agg = chunks[0] if na == 1 else jnp.concatenate(chunks, axis=1)
            rst = scale_ref[0, 0] * x_ref[...] + agg
            t = jnp.dot(rst, w0_ref[...],
                        preferred_element_type=jnp.float32) + b0_ref[...]
            t_s[blk] = t
            stats(0, t)

        @pl.when(p == 1)
        def _():
            a, cc = coefs(0, g0_ref[...], be0_ref[...])
            u = jnp.maximum(t_s[blk] * a + cc, 0.0)
            t2 = jnp.dot(u, w1_ref[...],
                         preferred_element_type=jnp.float32) + b1_ref[...]
            t_s[blk] = t2
            stats(1, t2)

        @pl.when(p == 2)
        def _():
            a, cc = coefs(1, ga_ref[...], ba_ref[...])
            v = jnp.maximum(t_s[blk] * a + cc, 0.0)
            t_s[blk] = v
            stats(2, v)

        @pl.when(p == 3)
        def _():
            a, cc = coefs(2, go_ref[...], bo_ref[...])
            xo = jnp.maximum(t_s[blk] * a + cc, 0.0)
            x_out[...] = xo
            if emit_cat:
                cat_ref = outs[1]
                cat_ref[0] = xo[:, :dh]
                cat_ref[1] = xo[:, dh:]

    out_specs = [
        pl.BlockSpec((_BR, do), lambda p, i: (jnp.where(p == 3, i, 0), 0)),
    ]
    out_shape = [jax.ShapeDtypeStruct((n, do), jnp.float32)]
    if emit_cat:
        out_specs.append(
            pl.BlockSpec((2, _BR, dh),
                         lambda p, i: (0, jnp.where(p == 3, i, 0), 0)))
        out_shape.append(jax.ShapeDtypeStruct((2, n, dh), jnp.float32))

    vec = lambda a: a.reshape(1, -1)
    return pl.pallas_call(
        body,
        grid=(4, _G),
        in_specs=[
            pl.BlockSpec(memory_space=pltpu.SMEM),
            pl.BlockSpec((_BR, d), lambda p, i: (jnp.where(p == 0, i, 0), 0)),
        ] + [
            pl.BlockSpec((2, _BR, 128),
                         lambda p, i: (0, jnp.where(p == 0, i, 0), 0))
            for _ in aggs
        ] + [
            pl.BlockSpec((d, do), lambda p, i: (0, 0)),
        ] + [
            pl.BlockSpec((1, do), lambda p, i: (0, 0)),
        ] * 3 + [
            pl.BlockSpec((do, do), lambda p, i: (0, 0)),
        ] + [
            pl.BlockSpec((1, do), lambda p, i: (0, 0)),
        ] * 5,
        out_specs=out_specs,
        out_shape=out_shape,
        scratch_shapes=[
            pltpu.VMEM((_N, do), jnp.float32),
            pltpu.VMEM((3, 2, do), jnp.float32),
        ],
    )(scale, x, *aggs, w0, vec(b0), vec(g0), vec(be0), w1, vec(b1), vec(ga),
      vec(ba), vec(go), vec(bo))


def kernel(h, edge_index,
           l0_W0, l0_b0, l0_g0, l0_be0, l0_W1, l0_b1, l0_ga, l0_ba, l0_go,
           l0_bo, l0_eps,
           l1_W0, l1_b0, l1_g0, l1_be0, l1_W1, l1_b1, l1_ga, l1_ba, l1_go,
           l1_bo, l1_eps):
    src2 = jnp.stack([edge_index[0], edge_index[0] + _N])
    dst = edge_index[1]

    params = [
        (l0_W0, l0_b0, l0_g0, l0_be0, l0_W1, l0_b1, l0_ga, l0_ba, l0_go,
         l0_bo, l0_eps),
        (l1_W0, l1_b0, l1_g0, l1_be0, l1_W1, l1_b1, l1_ga, l1_ba, l1_go,
         l1_bo, l1_eps),
    ]

    outs = [h]
    x = h
    tables = [(h, 0)]
    zeros = jnp.zeros((_NP, 128), jnp.float32)
    ntile = _NC * _NS
    dst3 = dst.reshape(ntile, _E // ntile // _CH, _CH)

    for i, (w0, b0, g0, be0, w1, b1, ga, ba, go, bo, eps) in enumerate(params):
        aggs = [_make_segsum()(t, src2[hh], dst3, zeros)
                for t, hh in tables]
        scale = (1.0 + eps).reshape(1, 1)
        if i == 0:
            x, cat = _tc_layer(x, aggs, scale, w0, b0, g0, be0, w1, b1, ga,
                               ba, go, bo, emit_cat=True)
            cat2 = cat.reshape(2 * _N, 128)
            tables = [(cat2, 0), (cat2, 1)]
        else:
            (x,) = _tc_layer(x, aggs, scale, w0, b0, g0, be0, w1, b1, ga,
                             ba, go, bo, emit_cat=False)
        outs.append(x)

    return tuple(outs)
